# baseline (speedup 1.0000x reference)
import functools

import jax
import jax.numpy as jnp
from jax import lax
from jax.experimental import pallas as pl
from jax.experimental.pallas import tpu as pltpu
from jax.experimental.pallas import tpu_sc as plsc

_SEQ = 200
_BATCH = 4096
_DIM = 64
_NC = 2
_NS = 16
_NW = _NC * _NS
_SEQS_PER_W = _BATCH // _NW
_LANES = 16
_VREGS_PER_ROW = _DIM // _LANES


def _make_sc_call():
    mesh = plsc.VectorSubcoreMesh(
        core_axis_name="c", subcore_axis_name="s",
        num_cores=_NC, num_subcores=_NS)

    @functools.partial(
        pl.kernel,
        out_type=jax.ShapeDtypeStruct((_BATCH, _SEQ, _DIM), jnp.float32),
        mesh=mesh,
        scratch_types=[
            pltpu.VMEM((2, _SEQ), jnp.int32),
            pltpu.VMEM((2, _SEQ, _DIM), jnp.float32),
            pltpu.VMEM((_SEQ, _DIM), jnp.float32),
            pltpu.SemaphoreType.DMA((2,)),
            pltpu.SemaphoreType.DMA((2,)),
        ],
        compiler_params=pltpu.CompilerParams(use_tc_tiling_on_sc=False),
    )
    def sc_embed(x_hbm, tok_hbm, pos_hbm, out_hbm, idx_v, rows_v, pos_v,
                 gsem, osem):
        wid = lax.axis_index("s") * _NC + lax.axis_index("c")
        seq0 = wid * _SEQS_PER_W
        pltpu.sync_copy(pos_hbm, pos_v)

        pltpu.sync_copy(x_hbm.at[seq0], idx_v.at[0])
        pltpu.async_copy(tok_hbm.at[idx_v.at[0]], rows_v.at[0], gsem.at[0])

        def chunk_body(i, carry):
            p = lax.rem(i, 2)
            q = 1 - p

            @pl.when(i + 1 < _SEQS_PER_W)
            def _():
                @pl.when(i >= 1)
                def _():
                    pltpu.make_async_copy(
                        rows_v.at[q], out_hbm.at[seq0], osem.at[q]).wait()
                pltpu.sync_copy(x_hbm.at[seq0 + i + 1], idx_v.at[q])
                pltpu.async_copy(tok_hbm.at[idx_v.at[q]], rows_v.at[q],
                                 gsem.at[q])

            pltpu.make_async_copy(tok_hbm.at[idx_v.at[p]], rows_v.at[p],
                                  gsem.at[p]).wait()

            @plsc.parallel_loop(0, _SEQ, 1, unroll=4)
            def _(r):
                for c in range(_VREGS_PER_ROW):
                    s = pl.ds(c * _LANES, _LANES)
                    plsc.addupdate(rows_v.at[p, r, s], pos_v[r, s])

            pltpu.async_copy(rows_v.at[p], out_hbm.at[seq0 + i], osem.at[p])
            return carry

        lax.fori_loop(0, _SEQS_PER_W, chunk_body, 0)

        for p in range(2):
            pltpu.make_async_copy(rows_v.at[p], out_hbm.at[seq0],
                                  osem.at[p]).wait()

    return sc_embed


_sc_embed = _make_sc_call()


@jax.jit
def kernel(x, token_table, pos_table):
    return _sc_embed(x, token_table, pos_table)


# speedup vs baseline: 1.2791x; 1.2791x over previous
import functools

import jax
import jax.numpy as jnp
from jax import lax
from jax.experimental import pallas as pl
from jax.experimental.pallas import tpu as pltpu
from jax.experimental.pallas import tpu_sc as plsc

_SEQ = 200
_BATCH = 4096
_DIM = 64
_NC = 2
_NS = 16
_NW = _NC * _NS
_SEQS_PER_W = _BATCH // _NW
_LANES = 16
_VREGS_PER_ROW = _DIM // _LANES
_ROW_BATCH = 25


def _make_sc_call():
    mesh = plsc.VectorSubcoreMesh(
        core_axis_name="c", subcore_axis_name="s",
        num_cores=_NC, num_subcores=_NS)

    @functools.partial(
        pl.kernel,
        out_type=jax.ShapeDtypeStruct((_BATCH, _SEQ, _DIM), jnp.float32),
        mesh=mesh,
        scratch_types=[
            pltpu.VMEM((2, _SEQ), jnp.int32),
            pltpu.VMEM((2, _SEQ, _DIM), jnp.float32),
            pltpu.VMEM((_SEQ, _DIM), jnp.float32),
            pltpu.SemaphoreType.DMA((2,)),
            pltpu.SemaphoreType.DMA((2,)),
        ],
        compiler_params=pltpu.CompilerParams(use_tc_tiling_on_sc=True),
    )
    def sc_embed(x_hbm, tok_hbm, pos_hbm, out_hbm, idx_v, rows_v, pos_v,
                 gsem, osem):
        wid = lax.axis_index("s") * _NC + lax.axis_index("c")
        seq0 = wid * _SEQS_PER_W
        pltpu.sync_copy(pos_hbm, pos_v)

        def gather_seq(i, p):
            pltpu.sync_copy(x_hbm.at[seq0 + i], idx_v.at[p])

            def group_body(g, carry):
                vec = idx_v[p, pl.ds(g * _LANES, _LANES)]
                base = g * _LANES
                for j in range(_LANES):
                    pltpu.async_copy(tok_hbm.at[vec[j]],
                                     rows_v.at[p, base + j], gsem.at[p])
                return carry
            lax.fori_loop(0, _SEQ // _LANES, group_body, 0)
            vec = idx_v[p, pl.ds(_SEQ - _LANES, _LANES)]
            for j in range(_LANES // 2, _LANES):
                pltpu.async_copy(tok_hbm.at[vec[j]],
                                 rows_v.at[p, _SEQ - _LANES + j], gsem.at[p])

        def drain_seq(p):
            pltpu.make_async_copy(out_hbm.at[seq0], rows_v.at[p],
                                  gsem.at[p]).wait()

        gather_seq(0, 0)

        def chunk_body(i, carry):
            p = lax.rem(i, 2)
            q = 1 - p

            @pl.when(i + 1 < _SEQS_PER_W)
            def _():
                @pl.when(i >= 1)
                def _():
                    pltpu.make_async_copy(
                        rows_v.at[q], out_hbm.at[seq0], osem.at[q]).wait()
                gather_seq(i + 1, q)

            drain_seq(p)

            @plsc.parallel_loop(0, _SEQ, 1, unroll=4)
            def _(r):
                for c in range(_VREGS_PER_ROW):
                    s = pl.ds(c * _LANES, _LANES)
                    plsc.addupdate(rows_v.at[p, r, s], pos_v[r, s])

            pltpu.async_copy(rows_v.at[p], out_hbm.at[seq0 + i], osem.at[p])
            return carry

        lax.fori_loop(0, _SEQS_PER_W, chunk_body, 0)

        for p in range(2):
            pltpu.make_async_copy(rows_v.at[p], out_hbm.at[seq0],
                                  osem.at[p]).wait()

    return sc_embed


_sc_embed = _make_sc_call()


@jax.jit
def kernel(x, token_table, pos_table):
    return _sc_embed(x, token_table, pos_table)
